# baseline (speedup 1.0000x reference)
import jax
import jax.numpy as jnp
import numpy as np
from jax.experimental import pallas as pl
from jax.experimental.pallas import tpu as pltpu
from jax.sharding import Mesh, PartitionSpec as P
from jax.experimental.shard_map import shard_map


def _cdiv(a, b):
    return (a + b - 1) // b


def _fused_mlp_kernel(x_ref, w1_ref, b1_ref, w2_ref, b2_ref, o_ref):
    x = x_ref[...].astype(jnp.bfloat16)
    h = jnp.dot(x, w1_ref[...], preferred_element_type=jnp.float32)
    h = jnp.maximum(h + b1_ref[...], 0.0).astype(jnp.bfloat16)
    y = jnp.dot(h, w2_ref[...], preferred_element_type=jnp.float32)
    o_ref[...] = (y + b2_ref[...]).astype(o_ref.dtype)


def _mlp_pallas(x, w1, b1, w2, b2):
    B, in_dim = x.shape
    hidden = w1.shape[1]
    out_dim = w2.shape[1]
    out_dtype = x.dtype

    bb = min(1024, max(((B + 7) // 8) * 8, 8))
    grid = (_cdiv(B, bb),)

    flops = 2 * B * (in_dim * hidden + hidden * out_dim)
    bytes_accessed = (x.size * x.dtype.itemsize
                      + (w1.size + w2.size) * 2
                      + (b1.size + b2.size) * 4
                      + B * out_dim * jnp.dtype(out_dtype).itemsize)
    cost = pl.CostEstimate(flops=flops, transcendentals=0,
                           bytes_accessed=bytes_accessed)

    return pl.pallas_call(
        _fused_mlp_kernel,
        out_shape=jax.ShapeDtypeStruct((B, out_dim), out_dtype),
        grid=grid,
        in_specs=[
            pl.BlockSpec((bb, in_dim), lambda i: (i, 0)),
            pl.BlockSpec((in_dim, hidden), lambda i: (0, 0)),
            pl.BlockSpec((1, hidden), lambda i: (0, 0)),
            pl.BlockSpec((hidden, out_dim), lambda i: (0, 0)),
            pl.BlockSpec((1, out_dim), lambda i: (0, 0)),
        ],
        out_specs=pl.BlockSpec((bb, out_dim), lambda i: (i, 0)),
        compiler_params=pltpu.CompilerParams(
            dimension_semantics=("parallel",),
            vmem_limit_bytes=48 * 1024 * 1024),
        cost_estimate=cost,
    )(x, w1, b1, w2, b2)


def kernel(x, w1, b1, w2, b2):
    hidden = w1.shape[1]
    out_dim = w2.shape[1]

    w1 = w1.astype(jnp.bfloat16)
    w2 = w2.astype(jnp.bfloat16)
    b1 = b1.astype(jnp.float32).reshape(1, hidden)
    b2 = b2.astype(jnp.float32).reshape(1, out_dim)

    devs = jax.devices()
    n_cores = 2 if (len(devs) >= 2 and x.shape[0] % 2 == 0) else 1
    if n_cores == 1:
        return _mlp_pallas(x, w1, b1, w2, b2)

    mesh = Mesh(np.array(devs[:2]), ("b",))
    sharded = shard_map(
        _mlp_pallas,
        mesh=mesh,
        in_specs=(P("b", None), P(None, None), P(None, None),
                  P(None, None), P(None, None)),
        out_specs=P("b", None),
        check_rep=False,
    )
    return sharded(x, w1, b1, w2, b2)


# speedup vs baseline: 5.8946x; 5.8946x over previous
import jax
import jax.numpy as jnp
import numpy as np
from jax.experimental import pallas as pl
from jax.experimental.pallas import tpu as pltpu
from jax.sharding import Mesh, PartitionSpec as P
from jax.experimental.shard_map import shard_map


def _cdiv(a, b):
    return (a + b - 1) // b


def _fused_mlp_kernel(x_ref, w1_ref, b1_ref, w2_ref, b2_ref, o_ref):
    x = x_ref[...].astype(jnp.bfloat16)
    h = jnp.dot(x, w1_ref[...], preferred_element_type=jnp.float32)
    h = jnp.maximum(h + b1_ref[...], 0.0).astype(jnp.bfloat16)
    y = jnp.dot(h, w2_ref[...], preferred_element_type=jnp.float32)
    o_ref[...] = (y + b2_ref[...]).astype(o_ref.dtype)


def _mlp_pallas(x, w1, b1, w2, b2):
    B, in_dim = x.shape
    hidden = w1.shape[1]
    out_dim = w2.shape[1]
    out_dtype = x.dtype

    bb = min(1024, max(((B + 7) // 8) * 8, 8))
    grid = (_cdiv(B, bb),)

    flops = 2 * B * (in_dim * hidden + hidden * out_dim)
    bytes_accessed = (x.size * x.dtype.itemsize
                      + (w1.size + w2.size) * 2
                      + (b1.size + b2.size) * 4
                      + B * out_dim * jnp.dtype(out_dtype).itemsize)
    cost = pl.CostEstimate(flops=flops, transcendentals=0,
                           bytes_accessed=bytes_accessed)

    return pl.pallas_call(
        _fused_mlp_kernel,
        out_shape=jax.ShapeDtypeStruct((B, out_dim), out_dtype),
        grid=grid,
        in_specs=[
            pl.BlockSpec((bb, in_dim), lambda i: (i, 0)),
            pl.BlockSpec((in_dim, hidden), lambda i: (0, 0)),
            pl.BlockSpec((1, hidden), lambda i: (0, 0)),
            pl.BlockSpec((hidden, out_dim), lambda i: (0, 0)),
            pl.BlockSpec((1, out_dim), lambda i: (0, 0)),
        ],
        out_specs=pl.BlockSpec((bb, out_dim), lambda i: (i, 0)),
        compiler_params=pltpu.CompilerParams(
            dimension_semantics=("parallel",),
            vmem_limit_bytes=48 * 1024 * 1024),
        cost_estimate=cost,
    )(x, w1, b1, w2, b2)


def kernel(x, w1, b1, w2, b2):
    hidden = w1.shape[1]
    out_dim = w2.shape[1]

    w1 = w1.astype(jnp.bfloat16)
    w2 = w2.astype(jnp.bfloat16)
    b1 = b1.astype(jnp.float32).reshape(1, hidden)
    b2 = b2.astype(jnp.float32).reshape(1, out_dim)

    return _mlp_pallas(x, w1, b1, w2, b2)
